# baseline (speedup 1.0000x reference)
import functools

import jax
import jax.numpy as jnp
from jax import lax
from jax.experimental import pallas as pl
from jax.experimental.pallas import tpu as pltpu
from jax.experimental.pallas import tpu_sc as plsc

N = 10000
E = 320000
D = 128

NC = 2
NS = 16
NW = NC * NS
EPT = 10240
E_P = NW * EPT
N_P = 10240
RPT = N_P // NS
ZR = 320

KD = 128
CHD = EPT // KD

KA = 64
CHA = EPT // KA
QCH = 40
NBUF = 3
U = 5

_mesh = plsc.VectorSubcoreMesh(core_axis_name="c", subcore_axis_name="s")


def _sc_degree_body(cols_hbm, ones_hbm, zeros_hbm, out_hbm, col_v, ones_v,
                    acc, sem):
    cid = lax.axis_index("c")
    sid = lax.axis_index("s")
    wid = cid * NS + sid
    pltpu.sync_copy(zeros_hbm, acc.at[pl.ds(sid * RPT, ZR)])
    pltpu.sync_copy(zeros_hbm, acc.at[pl.ds(sid * RPT + ZR, ZR)])
    pltpu.sync_copy(cols_hbm.at[wid], col_v)
    pltpu.sync_copy(ones_hbm, ones_v)
    plsc.subcore_barrier()

    @pl.loop(0, CHD)
    def _(j):
        pltpu.sync_copy(ones_v, acc.at[col_v.at[j]], add=True)

    plsc.subcore_barrier()
    pltpu.sync_copy(acc.at[pl.ds(sid * RPT, ZR)],
                    out_hbm.at[cid, pl.ds(sid * RPT, ZR)])
    pltpu.sync_copy(acc.at[pl.ds(sid * RPT + ZR, ZR)],
                    out_hbm.at[cid, pl.ds(sid * RPT + ZR, ZR)])


def _sc_aggregate_body(z_hbm, rows_hbm, cols_hbm, zeros_hbm, out_hbm,
                       row_v, col_v, gb0, gb1, gb2, acc, sm0, sm1, sm2):
    gbufs = [gb0, gb1, gb2]
    sems = [sm0, sm1, sm2]
    cid = lax.axis_index("c")
    sid = lax.axis_index("s")
    wid = cid * NS + sid
    pltpu.sync_copy(zeros_hbm, acc.at[pl.ds(sid * RPT, ZR)])
    pltpu.sync_copy(zeros_hbm, acc.at[pl.ds(sid * RPT + ZR, ZR)])
    plsc.subcore_barrier()

    for q in range(CHA // QCH):
        pltpu.sync_copy(rows_hbm.at[wid, pl.ds(q * QCH, QCH)], row_v)
        pltpu.sync_copy(cols_hbm.at[wid, pl.ds(q * QCH, QCH)], col_v)

        @pl.loop(0, QCH, step=U)
        def _(j):
            descs = [None] * U
            descs[0] = pltpu.async_copy(z_hbm.at[row_v.at[j]],
                                        gbufs[0], sems[0])
            descs[1] = pltpu.async_copy(z_hbm.at[row_v.at[j + 1]],
                                        gbufs[1], sems[1])
            for t in range(U):
                if t + 2 < U:
                    descs[t + 2] = pltpu.async_copy(
                        z_hbm.at[row_v.at[j + t + 2]],
                        gbufs[(t + 2) % NBUF], sems[(t + 2) % NBUF])
                descs[t].wait()
                pltpu.sync_copy(gbufs[t % NBUF], acc.at[col_v.at[j + t]],
                                add=True)

    plsc.subcore_barrier()
    pltpu.sync_copy(acc.at[pl.ds(sid * RPT, ZR)],
                    out_hbm.at[cid, pl.ds(sid * RPT, ZR)])
    pltpu.sync_copy(acc.at[pl.ds(sid * RPT + ZR, ZR)],
                    out_hbm.at[cid, pl.ds(sid * RPT + ZR, ZR)])


def _build_sc_degree(interpret=False):
    return functools.partial(
        pl.kernel,
        out_type=jax.ShapeDtypeStruct((NC, N_P, D), jnp.float32),
        mesh=_mesh,
        scratch_types=[
            pltpu.VMEM((CHD, KD), jnp.int32),
            pltpu.VMEM((KD, D), jnp.float32),
            pltpu.VMEM_SHARED((N_P, D), jnp.float32),
            pltpu.SemaphoreType.DMA,
        ],
        interpret=interpret,
    )(_sc_degree_body)


def _build_sc_aggregate(interpret=False):
    return functools.partial(
        pl.kernel,
        out_type=jax.ShapeDtypeStruct((NC, N_P, D), jnp.float32),
        mesh=_mesh,
        scratch_types=[
            pltpu.VMEM((QCH, KA), jnp.int32),
            pltpu.VMEM((QCH, KA), jnp.int32),
            pltpu.VMEM((KA, D), jnp.float32),
            pltpu.VMEM((KA, D), jnp.float32),
            pltpu.VMEM((KA, D), jnp.float32),
            pltpu.VMEM_SHARED((N_P, D), jnp.float32),
            pltpu.SemaphoreType.DMA,
            pltpu.SemaphoreType.DMA,
            pltpu.SemaphoreType.DMA,
        ],
        interpret=interpret,
    )(_sc_aggregate_body)


_sc_degree = _build_sc_degree()
_sc_aggregate = _build_sc_aggregate()


_RB = 1280
_GRID = N_P // _RB


def _u_block(deg_ref):
    d = deg_ref[0][:, 0:1] + deg_ref[1][:, 0:1] + 1.0
    return lax.rsqrt(d)


def _tc1_body(deg_ref, x_ref, w_ref, z_ref):
    u = _u_block(deg_ref)
    xw = jnp.dot(x_ref[...], w_ref[...], preferred_element_type=jnp.float32)
    z_ref[...] = u * xw


def _tc2_body(deg_ref, acc_ref, z1_ref, w_ref, b1_ref, z2_ref):
    u = _u_block(deg_ref)
    h = u * (acc_ref[0] + acc_ref[1] + z1_ref[...]) + b1_ref[...]
    h = jnp.maximum(h, 0.0)
    z2_ref[...] = u * jnp.dot(h, w_ref[...], preferred_element_type=jnp.float32)


def _tc3_body(deg_ref, acc_ref, z2_ref, b2_ref, out_ref):
    u = _u_block(deg_ref)
    out_ref[...] = u * (acc_ref[0] + acc_ref[1] + z2_ref[...]) + b2_ref[...]


_deg_spec = pl.BlockSpec((NC, _RB, D), lambda i: (0, i, 0))
_acc_spec = pl.BlockSpec((NC, _RB, D), lambda i: (0, i, 0))
_row_spec = pl.BlockSpec((_RB, D), lambda i: (i, 0))
_mat_spec = pl.BlockSpec((D, D), lambda i: (0, 0))
_vec_spec = pl.BlockSpec((1, D), lambda i: (0, 0))
_f32 = jnp.float32


def _tc1(deg, x_p, w):
    return pl.pallas_call(
        _tc1_body,
        grid=(_GRID,),
        in_specs=[_deg_spec, _row_spec, _mat_spec],
        out_specs=_row_spec,
        out_shape=jax.ShapeDtypeStruct((N_P, D), _f32),
    )(deg, x_p, w)


def _tc2(deg, acc, z1, w2, b1):
    return pl.pallas_call(
        _tc2_body,
        grid=(_GRID,),
        in_specs=[_deg_spec, _acc_spec, _row_spec, _mat_spec, _vec_spec],
        out_specs=_row_spec,
        out_shape=jax.ShapeDtypeStruct((N_P, D), _f32),
    )(deg, acc, z1, w2, b1)


def _tc3(deg, acc, z2, b2):
    return pl.pallas_call(
        _tc3_body,
        grid=(_GRID,),
        in_specs=[_deg_spec, _acc_spec, _row_spec, _vec_spec],
        out_specs=_row_spec,
        out_shape=jax.ShapeDtypeStruct((N_P, D), _f32),
    )(deg, acc, z2, b2)


def kernel(x, edge_index, W1, b1, W2, b2):
    row = edge_index[0].astype(jnp.int32)
    col = edge_index[1].astype(jnp.int32)
    npad = E_P - E
    row_p = jnp.concatenate([row, jnp.zeros((npad,), jnp.int32)])
    col_p = jnp.concatenate([col, jnp.full((npad,), N_P - 1, jnp.int32)])
    rows_a = row_p.reshape(NW, CHA, KA)
    cols_a = col_p.reshape(NW, CHA, KA)
    cols_d = col_p.reshape(NW, CHD, KD)

    x_p = jnp.concatenate([x, jnp.zeros((N_P - N, D), _f32)])
    onesD = jnp.ones((KD, D), _f32)
    zerosD = jnp.zeros((ZR, D), _f32)
    b1r = b1.reshape(1, D)
    b2r = b2.reshape(1, D)

    deg = _sc_degree(cols_d, onesD, zerosD)
    z1 = _tc1(deg, x_p, W1)
    a1 = _sc_aggregate(z1, rows_a, cols_a, zerosD)
    z2 = _tc2(deg, a1, z1, W2, b1r)
    a2 = _sc_aggregate(z2, rows_a, cols_a, zerosD)
    out = _tc3(deg, a2, z2, b2r)
    return out[:N]


# speedup vs baseline: 1.4191x; 1.4191x over previous
import functools

import jax
import jax.numpy as jnp
from jax import lax
from jax.experimental import pallas as pl
from jax.experimental.pallas import tpu as pltpu
from jax.experimental.pallas import tpu_sc as plsc

N = 10000
E = 320000
D = 128

NC = 2
NS = 16
NW = NC * NS
EPT = 10240
E_P = NW * EPT
N_P = 10240
RPT = N_P // NS
ZR = 320

KD = 128
CHD = EPT // KD

KA = 64
CH0 = 240
CH1 = 80
QCH = 40
NBUF = 2
U = 5
E0 = NS * CH0 * KA
E1 = NS * CH1 * KA

_mesh = plsc.VectorSubcoreMesh(core_axis_name="c", subcore_axis_name="s")


def _sc_degree_body(cols_hbm, ones_hbm, zeros_hbm, out_hbm, col_v, ones_v,
                    acc, sem):
    cid = lax.axis_index("c")
    sid = lax.axis_index("s")
    wid = cid * NS + sid
    pltpu.sync_copy(zeros_hbm, acc.at[pl.ds(sid * RPT, ZR)])
    pltpu.sync_copy(zeros_hbm, acc.at[pl.ds(sid * RPT + ZR, ZR)])
    pltpu.sync_copy(cols_hbm.at[wid], col_v)
    pltpu.sync_copy(ones_hbm, ones_v)
    plsc.subcore_barrier()

    @pl.loop(0, CHD)
    def _(j):
        pltpu.sync_copy(ones_v, acc.at[col_v.at[j]], add=True)

    plsc.subcore_barrier()
    pltpu.sync_copy(acc.at[pl.ds(sid * RPT, ZR)],
                    out_hbm.at[cid, pl.ds(sid * RPT, ZR)])
    pltpu.sync_copy(acc.at[pl.ds(sid * RPT + ZR, ZR)],
                    out_hbm.at[cid, pl.ds(sid * RPT + ZR, ZR)])


def _agg_core_loop(z_hbm, rows_hbm, cols_hbm, sid, row_v, col_v, gbufs,
                   sems, acc, n_chunks):
    for q in range(n_chunks // QCH):
        pltpu.sync_copy(rows_hbm.at[sid, pl.ds(q * QCH, QCH)], row_v)
        pltpu.sync_copy(cols_hbm.at[sid, pl.ds(q * QCH, QCH)], col_v)

        @pl.loop(0, QCH, step=U)
        def _(j):
            descs = [None] * U
            descs[0] = pltpu.async_copy(z_hbm.at[row_v.at[j]],
                                        gbufs[0], sems[0])
            for t in range(U):
                if t + 1 < U:
                    descs[t + 1] = pltpu.async_copy(
                        z_hbm.at[row_v.at[j + t + 1]],
                        gbufs[(t + 1) % NBUF], sems[(t + 1) % NBUF])
                descs[t].wait()
                pltpu.sync_copy(gbufs[t % NBUF], acc.at[col_v.at[j + t]],
                                add=True)


def _sc_aggregate_body(z_hbm, rows0_hbm, cols0_hbm, rows1_hbm, cols1_hbm,
                       zeros_hbm, out_hbm, row_v, col_v, gb0, gb1, acc,
                       sm0, sm1):
    gbufs = [gb0, gb1]
    sems = [sm0, sm1]
    cid = lax.axis_index("c")
    sid = lax.axis_index("s")
    pltpu.sync_copy(zeros_hbm, acc.at[pl.ds(sid * RPT, ZR)])
    pltpu.sync_copy(zeros_hbm, acc.at[pl.ds(sid * RPT + ZR, ZR)])
    plsc.subcore_barrier()

    @pl.when(cid == 0)
    def _():
        _agg_core_loop(z_hbm, rows0_hbm, cols0_hbm, sid, row_v, col_v,
                       gbufs, sems, acc, CH0)

    @pl.when(cid == 1)
    def _():
        _agg_core_loop(z_hbm, rows1_hbm, cols1_hbm, sid, row_v, col_v,
                       gbufs, sems, acc, CH1)

    plsc.subcore_barrier()
    pltpu.sync_copy(acc.at[pl.ds(sid * RPT, ZR)],
                    out_hbm.at[cid, pl.ds(sid * RPT, ZR)])
    pltpu.sync_copy(acc.at[pl.ds(sid * RPT + ZR, ZR)],
                    out_hbm.at[cid, pl.ds(sid * RPT + ZR, ZR)])


def _build_sc_degree(interpret=False):
    return functools.partial(
        pl.kernel,
        out_type=jax.ShapeDtypeStruct((NC, N_P, D), jnp.float32),
        mesh=_mesh,
        scratch_types=[
            pltpu.VMEM((CHD, KD), jnp.int32),
            pltpu.VMEM((KD, D), jnp.float32),
            pltpu.VMEM_SHARED((N_P, D), jnp.float32),
            pltpu.SemaphoreType.DMA,
        ],
        interpret=interpret,
    )(_sc_degree_body)


def _build_sc_aggregate(interpret=False):
    return functools.partial(
        pl.kernel,
        out_type=jax.ShapeDtypeStruct((NC, N_P, D), jnp.float32),
        mesh=_mesh,
        scratch_types=[
            pltpu.VMEM((QCH, KA), jnp.int32),
            pltpu.VMEM((QCH, KA), jnp.int32),
            pltpu.VMEM((KA, D), jnp.float32),
            pltpu.VMEM((KA, D), jnp.float32),
            pltpu.VMEM_SHARED((N_P, D), jnp.float32),
            pltpu.SemaphoreType.DMA,
            pltpu.SemaphoreType.DMA,
        ],
        interpret=interpret,
    )(_sc_aggregate_body)


_sc_degree = _build_sc_degree()
_sc_aggregate = _build_sc_aggregate()


_RB = 1280
_GRID = N_P // _RB


def _u_block(deg_ref):
    d = deg_ref[0][:, 0:1] + deg_ref[1][:, 0:1] + 1.0
    return lax.rsqrt(d)


def _tc1_body(deg_ref, x_ref, w_ref, z_ref):
    u = _u_block(deg_ref)
    xw = jnp.dot(x_ref[...], w_ref[...], preferred_element_type=jnp.float32)
    z_ref[...] = u * xw


def _tc2_body(deg_ref, acc_ref, z1_ref, w_ref, b1_ref, z2_ref):
    u = _u_block(deg_ref)
    h = u * (acc_ref[0] + acc_ref[1] + z1_ref[...]) + b1_ref[...]
    h = jnp.maximum(h, 0.0)
    z2_ref[...] = u * jnp.dot(h, w_ref[...], preferred_element_type=jnp.float32)


def _tc3_body(deg_ref, acc_ref, z2_ref, b2_ref, out_ref):
    u = _u_block(deg_ref)
    out_ref[...] = u * (acc_ref[0] + acc_ref[1] + z2_ref[...]) + b2_ref[...]


_deg_spec = pl.BlockSpec((NC, _RB, D), lambda i: (0, i, 0))
_acc_spec = pl.BlockSpec((NC, _RB, D), lambda i: (0, i, 0))
_row_spec = pl.BlockSpec((_RB, D), lambda i: (i, 0))
_mat_spec = pl.BlockSpec((D, D), lambda i: (0, 0))
_vec_spec = pl.BlockSpec((1, D), lambda i: (0, 0))
_f32 = jnp.float32


def _tc1(deg, x_p, w):
    return pl.pallas_call(
        _tc1_body,
        grid=(_GRID,),
        in_specs=[_deg_spec, _row_spec, _mat_spec],
        out_specs=_row_spec,
        out_shape=jax.ShapeDtypeStruct((N_P, D), _f32),
    )(deg, x_p, w)


def _tc2(deg, acc, z1, w2, b1):
    return pl.pallas_call(
        _tc2_body,
        grid=(_GRID,),
        in_specs=[_deg_spec, _acc_spec, _row_spec, _mat_spec, _vec_spec],
        out_specs=_row_spec,
        out_shape=jax.ShapeDtypeStruct((N_P, D), _f32),
    )(deg, acc, z1, w2, b1)


def _tc3(deg, acc, z2, b2):
    return pl.pallas_call(
        _tc3_body,
        grid=(_GRID,),
        in_specs=[_deg_spec, _acc_spec, _row_spec, _vec_spec],
        out_specs=_row_spec,
        out_shape=jax.ShapeDtypeStruct((N_P, D), _f32),
    )(deg, acc, z2, b2)


def kernel(x, edge_index, W1, b1, W2, b2):
    row = edge_index[0].astype(jnp.int32)
    col = edge_index[1].astype(jnp.int32)
    npad = E_P - E
    row_p = jnp.concatenate([row, jnp.zeros((npad,), jnp.int32)])
    col_p = jnp.concatenate([col, jnp.full((npad,), N_P - 1, jnp.int32)])
    rows0 = row_p[:E0].reshape(NS, CH0, KA)
    cols0 = col_p[:E0].reshape(NS, CH0, KA)
    rows1 = row_p[E0:].reshape(NS, CH1, KA)
    cols1 = col_p[E0:].reshape(NS, CH1, KA)
    cols_d = col_p.reshape(NW, CHD, KD)

    x_p = jnp.concatenate([x, jnp.zeros((N_P - N, D), _f32)])
    onesD = jnp.ones((KD, D), _f32)
    zerosD = jnp.zeros((ZR, D), _f32)
    b1r = b1.reshape(1, D)
    b2r = b2.reshape(1, D)

    deg = _sc_degree(cols_d, onesD, zerosD)
    z1 = _tc1(deg, x_p, W1)
    a1 = _sc_aggregate(z1, rows0, cols0, rows1, cols1, zerosD)
    z2 = _tc2(deg, a1, z1, W2, b1r)
    a2 = _sc_aggregate(z2, rows0, cols0, rows1, cols1, zerosD)
    out = _tc3(deg, a2, z2, b2r)
    return out[:N]
